# Initial kernel scaffold; baseline (speedup 1.0000x reference)
import jax
import jax.numpy as jnp
from jax import lax
from jax.experimental import pallas as pl
from jax.experimental.pallas import tpu as pltpu
from jax.experimental.pallas import tpu_sc as plsc

_NC = 2
_NS = 16
_NW = _NC * _NS
_L = 16

_N = 10000
_E = 320000
_D = 128

_N_PAD = 10240
_TRASH = _N
_CHW = 128
_CH = 80
_RPT = _N_PAD // _NS
_BR = 512


def _deg_body(src_hbm, dst_hbm, clus_hbm, hist_out, src_v, dst_v, clus_v, hist_v):
    c = lax.axis_index("c")
    s = lax.axis_index("s")
    wid = s * _NC + c
    pltpu.sync_copy(clus_hbm, clus_v)
    pltpu.sync_copy(src_hbm.at[wid], src_v)
    pltpu.sync_copy(dst_hbm.at[wid], dst_v)

    zeros16 = jnp.zeros((_L,), jnp.float32)

    @pl.loop(0, _N_PAD // _L)
    def _zero(i):
        hist_v[pl.ds(i * _L, _L)] = zeros16

    ones16 = jnp.ones((_L,), jnp.float32)

    @pl.loop(0, _CH)
    def _chunk(j):
        for k in range(_CHW // _L):
            sl = pl.ds(k * _L, _L)
            sidx = src_v[j, sl]
            didx = dst_v[j, sl]
            cs = plsc.load_gather(clus_v, [sidx])
            cd = plsc.load_gather(clus_v, [didx])
            plsc.addupdate_scatter(hist_v, [didx], ones16, mask=cs == cd)

    pltpu.sync_copy(hist_v, hist_out.at[wid])


def _agg_body(y_hbm, src_hbm, dst_hbm, clus_hbm, agg_out,
              src_v, dst2_v, clus_v, rows0, rows1, zero_v, agg_sh, sem0, sem1):
    c = lax.axis_index("c")
    s = lax.axis_index("s")
    wid = s * _NC + c

    pltpu.sync_copy(clus_hbm, clus_v)
    pltpu.sync_copy(src_hbm.at[wid], src_v)
    pltpu.sync_copy(dst_hbm.at[wid], dst2_v)

    zeros16 = jnp.zeros((_L,), jnp.float32)

    @pl.loop(0, 64)
    def _zbuf(i):
        for k in range(_D // _L):
            zero_v[i, pl.ds(k * _L, _L)] = zeros16

    @pl.loop(0, _RPT // 64)
    def _zstripe(i):
        pltpu.sync_copy(zero_v, agg_sh.at[pl.ds(s * _RPT + i * 64, 64)])

    trash16 = jnp.full((_L,), _TRASH, jnp.int32)

    @pl.loop(0, _CH)
    def _mask(j):
        for k in range(_CHW // _L):
            sl = pl.ds(k * _L, _L)
            sidx = src_v[j, sl]
            didx = dst2_v[j, sl]
            cs = plsc.load_gather(clus_v, [sidx])
            cd = plsc.load_gather(clus_v, [didx])
            dst2_v[j, sl] = jnp.where(cs == cd, didx, trash16)

    pltpu.async_copy(y_hbm.at[src_v.at[0]], rows0, sem0)
    plsc.subcore_barrier()

    @pl.loop(0, _CH // 2)
    def _pipe(i):
        j0 = 2 * i
        j1 = j0 + 1
        pltpu.async_copy(y_hbm.at[src_v.at[j1]], rows1, sem1)
        pltpu.make_async_copy(y_hbm.at[src_v.at[j0]], rows0, sem0).wait()
        pltpu.sync_copy(rows0, agg_sh.at[dst2_v.at[j0]], add=True)

        @pl.when(j1 + 1 < _CH)
        def _start_next():
            pltpu.async_copy(y_hbm.at[src_v.at[j1 + 1]], rows0, sem0)

        pltpu.make_async_copy(y_hbm.at[src_v.at[j1]], rows1, sem1).wait()
        pltpu.sync_copy(rows1, agg_sh.at[dst2_v.at[j1]], add=True)

    plsc.subcore_barrier()
    pltpu.sync_copy(agg_sh.at[pl.ds(s * _RPT, _RPT)],
                    agg_out.at[c, pl.ds(s * _RPT, _RPT)])


def _y_body(x_ref, w_ref, hist_ref, y_ref):
    deg = jnp.sum(hist_ref[...], axis=0) + 1.0
    dinv = lax.rsqrt(deg)
    xw = jnp.dot(x_ref[...], w_ref[...], preferred_element_type=jnp.float32)
    y_ref[...] = xw * dinv[:, None]


def _combine_body(agg_ref, y_ref, hist_ref, b_ref, o_ref):
    deg = jnp.sum(hist_ref[...], axis=0) + 1.0
    dinv = lax.rsqrt(deg)
    o_ref[...] = dinv[:, None] * (agg_ref[0] + agg_ref[1] + y_ref[...]) + b_ref[...]


def _sc_mesh():
    return plsc.VectorSubcoreMesh(core_axis_name="c", subcore_axis_name="s")


def _deg_call(src_p, dst_p, clus_p):
    f = pl.kernel(
        _deg_body,
        out_type=jax.ShapeDtypeStruct((_NW, _N_PAD), jnp.float32),
        mesh=_sc_mesh(),
        scratch_types=[
            pltpu.VMEM((_CH, _CHW), jnp.int32),
            pltpu.VMEM((_CH, _CHW), jnp.int32),
            pltpu.VMEM((_N_PAD,), jnp.int32),
            pltpu.VMEM((_N_PAD,), jnp.float32),
        ],
    )
    return f(src_p, dst_p, clus_p)


def _agg_call(y, src_p, dst_p, clus_p):
    f = pl.kernel(
        _agg_body,
        out_type=jax.ShapeDtypeStruct((_NC, _N_PAD, _D), jnp.float32),
        mesh=_sc_mesh(),
        scratch_types=[
            pltpu.VMEM((_CH, _CHW), jnp.int32),
            pltpu.VMEM((_CH, _CHW), jnp.int32),
            pltpu.VMEM((_N_PAD,), jnp.int32),
            pltpu.VMEM((_CHW, _D), jnp.float32),
            pltpu.VMEM((_CHW, _D), jnp.float32),
            pltpu.VMEM((64, _D), jnp.float32),
            pltpu.VMEM_SHARED((_N_PAD, _D), jnp.float32),
            pltpu.SemaphoreType.DMA,
            pltpu.SemaphoreType.DMA,
        ],
    )
    return f(y, src_p, dst_p, clus_p)


def _y_call(x_p, w, hist):
    return pl.pallas_call(
        _y_body,
        grid=(_N_PAD // _BR,),
        in_specs=[
            pl.BlockSpec((_BR, _D), lambda i: (i, 0)),
            pl.BlockSpec((_D, _D), lambda i: (0, 0)),
            pl.BlockSpec((_NW, _BR), lambda i: (0, i)),
        ],
        out_specs=pl.BlockSpec((_BR, _D), lambda i: (i, 0)),
        out_shape=jax.ShapeDtypeStruct((_N_PAD, _D), jnp.float32),
    )(x_p, w, hist)


def _combine_call(agg, y, hist, b2):
    return pl.pallas_call(
        _combine_body,
        grid=(_N_PAD // _BR,),
        in_specs=[
            pl.BlockSpec((_NC, _BR, _D), lambda i: (0, i, 0)),
            pl.BlockSpec((_BR, _D), lambda i: (i, 0)),
            pl.BlockSpec((_NW, _BR), lambda i: (0, i)),
            pl.BlockSpec((1, _D), lambda i: (0, 0)),
        ],
        out_specs=pl.BlockSpec((_BR, _D), lambda i: (i, 0)),
        out_shape=jax.ShapeDtypeStruct((_N_PAD, _D), jnp.float32),
    )(agg, y, hist, b2)


def kernel(X, W, b, cluster_assignment, full_edge_index):
    n, d = X.shape
    src = full_edge_index[0]
    dst = full_edge_index[1]
    e = src.shape[0]
    pad_e = _NW * _CH * _CHW - e

    src_p = jnp.concatenate(
        [src, jnp.zeros((pad_e,), jnp.int32)]).reshape(_NW, _CH, _CHW)
    dst_p = jnp.concatenate(
        [dst, jnp.full((pad_e,), _TRASH, jnp.int32)]).reshape(_NW, _CH, _CHW)
    clus_p = jnp.concatenate(
        [cluster_assignment, jnp.full((_N_PAD - n,), -1, jnp.int32)])
    x_p = jnp.pad(X, ((0, _N_PAD - n), (0, 0)))

    hist = _deg_call(src_p, dst_p, clus_p)
    y = _y_call(x_p, W, hist)
    agg = _agg_call(y, src_p, dst_p, clus_p)
    out = _combine_call(agg, y, hist, b.reshape(1, d))
    return out[:n]


# speedup vs baseline: 19.5540x; 19.5540x over previous
import jax
import jax.numpy as jnp
from jax import lax
from jax.experimental import pallas as pl
from jax.experimental.pallas import tpu as pltpu
from jax.experimental.pallas import tpu_sc as plsc

_NC = 2
_NS = 16
_NW = _NC * _NS
_L = 16

_N = 10000
_E = 320000
_D = 128
_DH = _D // 2

_N_PAD = 10240
_TRASH = _N
_E_PAD = 327680

_CHD = 160
_CWD = 64
_CHA = 160
_CWA = 128

_RPT = _N_PAD // _NS
_BR = 512


def _deg_body(src_hbm, dst_hbm, clus_hbm, hist_out, dst2_out,
              src_v, dst_v, clus_v, hist_v):
    c = lax.axis_index("c")
    s = lax.axis_index("s")
    wid = s * _NC + c
    pltpu.sync_copy(clus_hbm, clus_v)
    pltpu.sync_copy(src_hbm.at[wid], src_v)
    pltpu.sync_copy(dst_hbm.at[wid], dst_v)

    zeros16 = jnp.zeros((_L,), jnp.float32)

    @pl.loop(0, _N_PAD // _L)
    def _zero(i):
        hist_v[pl.ds(i * _L, _L)] = zeros16

    ones16 = jnp.ones((_L,), jnp.float32)
    trash16 = jnp.full((_L,), _TRASH, jnp.int32)

    @pl.loop(0, _CHD)
    def _chunk(j):
        for k in range(_CWD // _L):
            sl = pl.ds(k * _L, _L)
            sidx = src_v[j, sl]
            didx = dst_v[j, sl]
            cs = plsc.load_gather(clus_v, [sidx])
            cd = plsc.load_gather(clus_v, [didx])
            m = cs == cd
            plsc.addupdate_scatter(hist_v, [didx], ones16, mask=m)
            dst_v[j, sl] = jnp.where(m, didx, trash16)

    pltpu.sync_copy(hist_v, hist_out.at[wid])
    pltpu.sync_copy(dst_v, dst2_out.at[wid])


def _agg_body(y_hbm, src_hbm, dst2_hbm, agg_out,
              src_v, dst2_v, rows0, rows1, zero_v, agg_sh, sem0, sem1):
    c = lax.axis_index("c")
    s = lax.axis_index("s")

    pltpu.sync_copy(src_hbm.at[s], src_v)
    pltpu.sync_copy(dst2_hbm.at[s], dst2_v)

    zeros16 = jnp.zeros((_L,), jnp.float32)

    @pl.loop(0, 16)
    def _zbuf(i):
        for k in range(_DH // _L):
            zero_v[i, pl.ds(k * _L, _L)] = zeros16

    @pl.loop(0, _RPT // 16)
    def _zstripe(i):
        pltpu.sync_copy(zero_v, agg_sh.at[pl.ds(s * _RPT + i * 16, 16)])

    ytab = y_hbm.at[c]

    pltpu.async_copy(ytab.at[src_v.at[0]], rows0, sem0)
    plsc.subcore_barrier()

    @pl.loop(0, _CHA // 2)
    def _pipe(i):
        j0 = 2 * i
        j1 = j0 + 1
        pltpu.async_copy(ytab.at[src_v.at[j1]], rows1, sem1)
        pltpu.make_async_copy(ytab.at[src_v.at[j0]], rows0, sem0).wait()
        pltpu.sync_copy(rows0, agg_sh.at[dst2_v.at[j0]], add=True)

        @pl.when(j1 + 1 < _CHA)
        def _start_next():
            pltpu.async_copy(ytab.at[src_v.at[j1 + 1]], rows0, sem0)

        pltpu.make_async_copy(ytab.at[src_v.at[j1]], rows1, sem1).wait()
        pltpu.sync_copy(rows1, agg_sh.at[dst2_v.at[j1]], add=True)

    plsc.subcore_barrier()
    pltpu.sync_copy(agg_sh.at[pl.ds(s * _RPT, _RPT)],
                    agg_out.at[c, pl.ds(s * _RPT, _RPT)])


def _y_body(x_ref, w_ref, hist_ref, y_ref):
    deg = jnp.sum(hist_ref[...], axis=0) + 1.0
    dinv = lax.rsqrt(deg)
    xw = jnp.dot(x_ref[...], w_ref[...], preferred_element_type=jnp.float32)
    y = xw * dinv[:, None]
    y_ref[0] = y[:, :_DH]
    y_ref[1] = y[:, _DH:]


def _combine_body(agg_ref, y_ref, hist_ref, b_ref, o_ref):
    deg = jnp.sum(hist_ref[...], axis=0) + 1.0
    dinv = lax.rsqrt(deg)
    tot = jnp.concatenate([agg_ref[0] + y_ref[0], agg_ref[1] + y_ref[1]], axis=1)
    o_ref[...] = dinv[:, None] * tot + b_ref[...]


def _sc_mesh():
    return plsc.VectorSubcoreMesh(core_axis_name="c", subcore_axis_name="s")


def _deg_call(src_p, dst_p, clus_p):
    f = pl.kernel(
        _deg_body,
        out_type=(
            jax.ShapeDtypeStruct((_NW, _N_PAD), jnp.float32),
            jax.ShapeDtypeStruct((_NW, _CHD, _CWD), jnp.int32),
        ),
        mesh=_sc_mesh(),
        scratch_types=[
            pltpu.VMEM((_CHD, _CWD), jnp.int32),
            pltpu.VMEM((_CHD, _CWD), jnp.int32),
            pltpu.VMEM((_N_PAD,), jnp.int32),
            pltpu.VMEM((_N_PAD,), jnp.float32),
        ],
        compiler_params=pltpu.CompilerParams(
            needs_layout_passes=False, use_tc_tiling_on_sc=False),
    )
    return f(src_p, dst_p, clus_p)


def _agg_call(y2, src_a, dst2_a):
    f = pl.kernel(
        _agg_body,
        out_type=jax.ShapeDtypeStruct((_NC, _N_PAD, _DH), jnp.float32),
        mesh=_sc_mesh(),
        scratch_types=[
            pltpu.VMEM((_CHA, _CWA), jnp.int32),
            pltpu.VMEM((_CHA, _CWA), jnp.int32),
            pltpu.VMEM((_CWA, _DH), jnp.float32),
            pltpu.VMEM((_CWA, _DH), jnp.float32),
            pltpu.VMEM((16, _DH), jnp.float32),
            pltpu.VMEM_SHARED((_N_PAD, _DH), jnp.float32),
            pltpu.SemaphoreType.DMA,
            pltpu.SemaphoreType.DMA,
        ],
        compiler_params=pltpu.CompilerParams(
            needs_layout_passes=False, use_tc_tiling_on_sc=False),
    )
    return f(y2, src_a, dst2_a)


def _y_call(x_p, w, hist):
    return pl.pallas_call(
        _y_body,
        grid=(_N_PAD // _BR,),
        in_specs=[
            pl.BlockSpec((_BR, _D), lambda i: (i, 0)),
            pl.BlockSpec((_D, _D), lambda i: (0, 0)),
            pl.BlockSpec((_NW, _BR), lambda i: (0, i)),
        ],
        out_specs=pl.BlockSpec((_NC, _BR, _DH), lambda i: (0, i, 0)),
        out_shape=jax.ShapeDtypeStruct((_NC, _N_PAD, _DH), jnp.float32),
    )(x_p, w, hist)


def _combine_call(agg, y2, hist, b2):
    return pl.pallas_call(
        _combine_body,
        grid=(_N_PAD // _BR,),
        in_specs=[
            pl.BlockSpec((_NC, _BR, _DH), lambda i: (0, i, 0)),
            pl.BlockSpec((_NC, _BR, _DH), lambda i: (0, i, 0)),
            pl.BlockSpec((_NW, _BR), lambda i: (0, i)),
            pl.BlockSpec((1, _D), lambda i: (0, 0)),
        ],
        out_specs=pl.BlockSpec((_BR, _D), lambda i: (i, 0)),
        out_shape=jax.ShapeDtypeStruct((_N_PAD, _D), jnp.float32),
    )(agg, y2, hist, b2)


def kernel(X, W, b, cluster_assignment, full_edge_index):
    n, d = X.shape
    src = full_edge_index[0]
    dst = full_edge_index[1]
    e = src.shape[0]
    pad_e = _E_PAD - e

    src_p = jnp.concatenate(
        [src, jnp.zeros((pad_e,), jnp.int32)]).reshape(_NW, _CHD, _CWD)
    dst_p = jnp.concatenate(
        [dst, jnp.full((pad_e,), _TRASH, jnp.int32)]).reshape(_NW, _CHD, _CWD)
    clus_p = jnp.concatenate(
        [cluster_assignment, jnp.full((_N_PAD - n,), -1, jnp.int32)])
    x_p = jnp.pad(X, ((0, _N_PAD - n), (0, 0)))

    hist, dst2_p = _deg_call(src_p, dst_p, clus_p)
    y2 = _y_call(x_p, W, hist)
    agg = _agg_call(y2,
                    src_p.reshape(_NS, _CHA, _CWA),
                    dst2_p.reshape(_NS, _CHA, _CWA))
    out = _combine_call(agg, y2, hist, b.reshape(1, d))
    return out[:n]


# speedup vs baseline: 51.0163x; 2.6090x over previous
import jax
import jax.numpy as jnp
from jax import lax
from jax.experimental import pallas as pl
from jax.experimental.pallas import tpu as pltpu
from jax.experimental.pallas import tpu_sc as plsc

_NC = 2
_NS = 16
_NW = _NC * _NS
_L = 16

_N = 10000
_E = 320000
_D = 128
_DH = _D // 2

_N_PAD = 10240
_TRASH = _N
_E_PAD = 327680
_EPT = _E_PAD // _NW

_CHD = 160
_CWD = 64
_CWA = 128
_CHA = _EPT // _CWA

_RPT = _N_PAD // _NS
_BR = 512


def _deg_body(src_hbm, dst_hbm, clus_hbm, hist_out, csrc_out, cdst_out, cnt_out,
              src_v, dst_v, clus_v, hist_v, csrc_v, cdst_v, cnt_v):
    c = lax.axis_index("c")
    s = lax.axis_index("s")
    wid = s * _NC + c
    pltpu.sync_copy(clus_hbm, clus_v)
    pltpu.sync_copy(src_hbm.at[wid], src_v)
    pltpu.sync_copy(dst_hbm.at[wid], dst_v)

    zeros16 = jnp.zeros((_L,), jnp.float32)

    @pl.loop(0, _N_PAD // _L)
    def _zero(i):
        hist_v[pl.ds(i * _L, _L)] = zeros16

    zeros16i = jnp.zeros((_L,), jnp.int32)
    trash16 = jnp.full((_L,), _TRASH, jnp.int32)

    @pl.loop(0, _EPT // _L)
    def _pre(i):
        csrc_v[pl.ds(i * _L, _L)] = zeros16i
        cdst_v[pl.ds(i * _L, _L)] = trash16

    ones16 = jnp.ones((_L,), jnp.float32)

    @pl.loop(0, _CHD, init_carry=jnp.int32(0))
    def _chunk(j, off):
        for k in range(_CWD // _L):
            sl = pl.ds(k * _L, _L)
            sidx = src_v[j, sl]
            didx = dst_v[j, sl]
            cs = plsc.load_gather(clus_v, [sidx])
            cd = plsc.load_gather(clus_v, [didx])
            m = cs == cd
            plsc.addupdate_scatter(hist_v, [didx], ones16, mask=m)
            plsc.store_compressed(csrc_v.at[pl.ds(off, _L)], sidx, mask=m)
            plsc.store_compressed(cdst_v.at[pl.ds(off, _L)], didx, mask=m)
            off = off + plsc.all_reduce_population_count(m)[0]
        return off

    off = _chunk
    nch = (off + _CWA - 1) // _CWA
    cnt_v[...] = jnp.full((_L,), nch, jnp.int32)

    pltpu.sync_copy(hist_v, hist_out.at[wid])
    pltpu.sync_copy(csrc_v.at[pl.ds(0, _EPT)], csrc_out.at[wid])
    pltpu.sync_copy(cdst_v.at[pl.ds(0, _EPT)], cdst_out.at[wid])
    pltpu.sync_copy(cnt_v, cnt_out.at[wid])


def _agg_body(y_hbm, csrc_hbm, cdst_hbm, cnt_hbm, agg_out,
              src_v, dst_v, cnt_v, rows0, rows1, zero_v, agg_sh, sem0, sem1):
    c = lax.axis_index("c")
    s = lax.axis_index("s")

    zeros16 = jnp.zeros((_L,), jnp.float32)

    @pl.loop(0, 16)
    def _zbuf(i):
        for k in range(_DH // _L):
            zero_v[i, pl.ds(k * _L, _L)] = zeros16

    @pl.loop(0, _RPT // 16)
    def _zstripe(i):
        pltpu.sync_copy(zero_v, agg_sh.at[pl.ds(s * _RPT + i * 16, 16)])

    ytab = y_hbm.at[c]

    plsc.subcore_barrier()

    for r_i in range(2):
        r = 2 * s + r_i
        pltpu.sync_copy(cnt_hbm.at[r], cnt_v)
        pltpu.sync_copy(csrc_hbm.at[r], src_v)
        pltpu.sync_copy(cdst_hbm.at[r], dst_v)
        nch = jnp.maximum(cnt_v[...][0], 1)
        npair = (nch + 1) // 2
        last = 2 * npair

        pltpu.async_copy(ytab.at[src_v.at[0]], rows0, sem0)

        @pl.loop(0, npair)
        def _pipe(i):
            j0 = 2 * i
            j1 = j0 + 1
            pltpu.async_copy(ytab.at[src_v.at[j1]], rows1, sem1)
            pltpu.make_async_copy(ytab.at[src_v.at[j0]], rows0, sem0).wait()
            pltpu.sync_copy(rows0, agg_sh.at[dst_v.at[j0]], add=True)

            @pl.when(j1 + 1 < last)
            def _start_next():
                pltpu.async_copy(ytab.at[src_v.at[j1 + 1]], rows0, sem0)

            pltpu.make_async_copy(ytab.at[src_v.at[j1]], rows1, sem1).wait()
            pltpu.sync_copy(rows1, agg_sh.at[dst_v.at[j1]], add=True)

    plsc.subcore_barrier()
    pltpu.sync_copy(agg_sh.at[pl.ds(s * _RPT, _RPT)],
                    agg_out.at[c, pl.ds(s * _RPT, _RPT)])


def _y_body(x_ref, w_ref, hist_ref, y_ref):
    deg = jnp.sum(hist_ref[...], axis=0) + 1.0
    dinv = lax.rsqrt(deg)
    xw = jnp.dot(x_ref[...], w_ref[...], preferred_element_type=jnp.float32)
    y = xw * dinv[:, None]
    y_ref[0] = y[:, :_DH]
    y_ref[1] = y[:, _DH:]


def _combine_body(agg_ref, y_ref, hist_ref, b_ref, o_ref):
    deg = jnp.sum(hist_ref[...], axis=0) + 1.0
    dinv = lax.rsqrt(deg)
    tot = jnp.concatenate([agg_ref[0] + y_ref[0], agg_ref[1] + y_ref[1]], axis=1)
    o_ref[...] = dinv[:, None] * tot + b_ref[...]


def _sc_mesh():
    return plsc.VectorSubcoreMesh(core_axis_name="c", subcore_axis_name="s")


def _deg_call(src_p, dst_p, clus_p):
    f = pl.kernel(
        _deg_body,
        out_type=(
            jax.ShapeDtypeStruct((_NW, _N_PAD), jnp.float32),
            jax.ShapeDtypeStruct((_NW, _EPT), jnp.int32),
            jax.ShapeDtypeStruct((_NW, _EPT), jnp.int32),
            jax.ShapeDtypeStruct((_NW, _L), jnp.int32),
        ),
        mesh=_sc_mesh(),
        scratch_types=[
            pltpu.VMEM((_CHD, _CWD), jnp.int32),
            pltpu.VMEM((_CHD, _CWD), jnp.int32),
            pltpu.VMEM((_N_PAD,), jnp.int32),
            pltpu.VMEM((_N_PAD,), jnp.float32),
            pltpu.VMEM((_EPT + _L,), jnp.int32),
            pltpu.VMEM((_EPT + _L,), jnp.int32),
            pltpu.VMEM((_L,), jnp.int32),
        ],
        compiler_params=pltpu.CompilerParams(
            needs_layout_passes=False, use_tc_tiling_on_sc=False),
    )
    return f(src_p, dst_p, clus_p)


def _agg_call(y2, csrc_a, cdst_a, cnt):
    f = pl.kernel(
        _agg_body,
        out_type=jax.ShapeDtypeStruct((_NC, _N_PAD, _DH), jnp.float32),
        mesh=_sc_mesh(),
        scratch_types=[
            pltpu.VMEM((_CHA, _CWA), jnp.int32),
            pltpu.VMEM((_CHA, _CWA), jnp.int32),
            pltpu.VMEM((_L,), jnp.int32),
            pltpu.VMEM((_CWA, _DH), jnp.float32),
            pltpu.VMEM((_CWA, _DH), jnp.float32),
            pltpu.VMEM((16, _DH), jnp.float32),
            pltpu.VMEM_SHARED((_N_PAD, _DH), jnp.float32),
            pltpu.SemaphoreType.DMA,
            pltpu.SemaphoreType.DMA,
        ],
        compiler_params=pltpu.CompilerParams(
            needs_layout_passes=False, use_tc_tiling_on_sc=False),
    )
    return f(y2, csrc_a, cdst_a, cnt)


def _y_call(x_p, w, hist):
    return pl.pallas_call(
        _y_body,
        grid=(_N_PAD // _BR,),
        in_specs=[
            pl.BlockSpec((_BR, _D), lambda i: (i, 0)),
            pl.BlockSpec((_D, _D), lambda i: (0, 0)),
            pl.BlockSpec((_NW, _BR), lambda i: (0, i)),
        ],
        out_specs=pl.BlockSpec((_NC, _BR, _DH), lambda i: (0, i, 0)),
        out_shape=jax.ShapeDtypeStruct((_NC, _N_PAD, _DH), jnp.float32),
    )(x_p, w, hist)


def _combine_call(agg, y2, hist, b2):
    return pl.pallas_call(
        _combine_body,
        grid=(_N_PAD // _BR,),
        in_specs=[
            pl.BlockSpec((_NC, _BR, _DH), lambda i: (0, i, 0)),
            pl.BlockSpec((_NC, _BR, _DH), lambda i: (0, i, 0)),
            pl.BlockSpec((_NW, _BR), lambda i: (0, i)),
            pl.BlockSpec((1, _D), lambda i: (0, 0)),
        ],
        out_specs=pl.BlockSpec((_BR, _D), lambda i: (i, 0)),
        out_shape=jax.ShapeDtypeStruct((_N_PAD, _D), jnp.float32),
    )(agg, y2, hist, b2)


def kernel(X, W, b, cluster_assignment, full_edge_index):
    n, d = X.shape
    src = full_edge_index[0]
    dst = full_edge_index[1]
    e = src.shape[0]
    pad_e = _E_PAD - e

    src_p = jnp.concatenate(
        [src, jnp.zeros((pad_e,), jnp.int32)]).reshape(_NW, _CHD, _CWD)
    dst_p = jnp.concatenate(
        [dst, jnp.full((pad_e,), _TRASH, jnp.int32)]).reshape(_NW, _CHD, _CWD)
    clus_p = jnp.concatenate(
        [cluster_assignment, jnp.full((_N_PAD - n,), -1, jnp.int32)])
    x_p = jnp.pad(X, ((0, _N_PAD - n), (0, 0)))

    hist, csrc, cdst, cnt = _deg_call(src_p, dst_p, clus_p)
    y2 = _y_call(x_p, W, hist)
    agg = _agg_call(y2,
                    csrc.reshape(_NW, _CHA, _CWA),
                    cdst.reshape(_NW, _CHA, _CWA),
                    cnt)
    out = _combine_call(agg, y2, hist, b.reshape(1, d))
    return out[:n]


# speedup vs baseline: 61.2662x; 1.2009x over previous
import jax
import jax.numpy as jnp
from jax import lax
from jax.experimental import pallas as pl
from jax.experimental.pallas import tpu as pltpu
from jax.experimental.pallas import tpu_sc as plsc

_NC = 2
_NS = 16
_NW = _NC * _NS
_L = 16

_N = 10000
_E = 320000
_D = 128
_DH = _D // 2

_N_PAD = 10240
_TRASH = _N
_EPT = _E // _NW

_CWA = 128
_CREG = 81
_RSZ = _CREG * _CWA

_RPT = _N_PAD // _NS
_BR = 400


def _deg_body(fei_hbm, clus_hbm, hist_out, csrc_out, cdst_out, cnt_out,
              src_v, dst_v, clus_v, hist_v, csrc_v, cdst_v, cnt_v):
    c = lax.axis_index("c")
    s = lax.axis_index("s")
    wid = s * _NC + c
    pltpu.sync_copy(clus_hbm, clus_v)
    pltpu.sync_copy(fei_hbm.at[0, pl.ds(wid * _EPT, _EPT)], src_v)
    pltpu.sync_copy(fei_hbm.at[1, pl.ds(wid * _EPT, _EPT)], dst_v)

    zeros16 = jnp.zeros((_L,), jnp.float32)

    @pl.loop(0, _N // _L)
    def _zero(i):
        hist_v[pl.ds(i * _L, _L)] = zeros16

    zeros16i = jnp.zeros((_L,), jnp.int32)
    trash16 = jnp.full((_L,), _TRASH, jnp.int32)

    @pl.loop(0, _RSZ // _L)
    def _pre(i):
        csrc_v[pl.ds(i * _L, _L)] = zeros16i
        cdst_v[pl.ds(i * _L, _L)] = trash16

    ones16 = jnp.ones((_L,), jnp.float32)

    @pl.loop(0, _EPT // _L, init_carry=jnp.int32(0))
    def _group(g, off):
        sl = pl.ds(g * _L, _L)
        sidx = src_v[sl]
        didx = dst_v[sl]
        cs = plsc.load_gather(clus_v, [sidx])
        cd = plsc.load_gather(clus_v, [didx])
        m = cs == cd
        plsc.addupdate_scatter(hist_v, [didx], ones16, mask=m)
        plsc.store_compressed(csrc_v.at[pl.ds(off, _L)], sidx, mask=m)
        plsc.store_compressed(cdst_v.at[pl.ds(off, _L)], didx, mask=m)
        return off + plsc.all_reduce_population_count(m)[0]

    off = _group
    nch = (off + _CWA - 1) // _CWA
    cnt_v[...] = jnp.full((_L,), nch, jnp.int32)

    pltpu.sync_copy(hist_v, hist_out.at[wid])
    pltpu.sync_copy(csrc_v.at[pl.ds(0, _RSZ)], csrc_out.at[wid])
    pltpu.sync_copy(cdst_v.at[pl.ds(0, _RSZ)], cdst_out.at[wid])
    pltpu.sync_copy(cnt_v, cnt_out.at[wid])


def _agg_body(y_hbm, csrc_hbm, cdst_hbm, cnt_hbm, agg_out,
              src_v, dst_v, cnt_v, rows0, rows1, zero_v, agg_sh, sem0, sem1):
    c = lax.axis_index("c")
    s = lax.axis_index("s")

    zeros16 = jnp.zeros((_L,), jnp.float32)

    @pl.loop(0, 16)
    def _zbuf(i):
        for k in range(_DH // _L):
            zero_v[i, pl.ds(k * _L, _L)] = zeros16

    @pl.loop(0, _RPT // 16)
    def _zstripe(i):
        pltpu.sync_copy(zero_v, agg_sh.at[pl.ds(s * _RPT + i * 16, 16)])

    ytab = y_hbm.at[c]
    pltpu.sync_copy(cnt_hbm.at[s], cnt_v)
    pltpu.sync_copy(csrc_hbm.at[2 * s], src_v.at[0])
    pltpu.sync_copy(csrc_hbm.at[2 * s + 1], src_v.at[1])
    pltpu.sync_copy(cdst_hbm.at[2 * s], dst_v.at[0])
    pltpu.sync_copy(cdst_hbm.at[2 * s + 1], dst_v.at[1])
    n0 = cnt_v[0][0]
    n1 = cnt_v[1][0]
    tot = n0 + n1
    npair = (jnp.maximum(tot, 1) + 1) // 2
    last = 2 * npair

    def chref(arr, j):
        in0 = j < n0
        inr = j < tot
        r_sel = jnp.where(in0 | (~inr), 0, 1)
        ch = jnp.where(in0, j, jnp.where(inr, j - n0, _CREG - 1))
        return arr.at[r_sel, ch]

    plsc.subcore_barrier()
    pltpu.async_copy(ytab.at[chref(src_v, 0)], rows0, sem0)

    @pl.loop(0, npair)
    def _pipe(i):
        j0 = 2 * i
        j1 = j0 + 1
        pltpu.async_copy(ytab.at[chref(src_v, j1)], rows1, sem1)
        pltpu.make_async_copy(ytab.at[chref(src_v, j0)], rows0, sem0).wait()
        pltpu.sync_copy(rows0, agg_sh.at[chref(dst_v, j0)], add=True)

        @pl.when(j1 + 1 < last)
        def _start_next():
            pltpu.async_copy(ytab.at[chref(src_v, j1 + 1)], rows0, sem0)

        pltpu.make_async_copy(ytab.at[chref(src_v, j1)], rows1, sem1).wait()
        pltpu.sync_copy(rows1, agg_sh.at[chref(dst_v, j1)], add=True)

    plsc.subcore_barrier()
    pltpu.sync_copy(agg_sh.at[pl.ds(s * _RPT, _RPT)],
                    agg_out.at[c, pl.ds(s * _RPT, _RPT)])


def _dinv_body(hist_ref, dinv_ref):
    deg = jnp.sum(hist_ref[...], axis=0) + 1.0
    dinv_ref[...] = lax.rsqrt(deg)[:, None]


def _y_body(x_ref, w_ref, dinv_ref, y_ref):
    xw = jnp.dot(x_ref[...], w_ref[...], preferred_element_type=jnp.float32)
    y = xw * dinv_ref[...]
    y_ref[0] = y[:, :_DH]
    y_ref[1] = y[:, _DH:]


def _combine_body(agg_ref, y_ref, dinv_ref, b_ref, o_ref):
    tot = jnp.concatenate([agg_ref[0] + y_ref[0], agg_ref[1] + y_ref[1]], axis=1)
    o_ref[...] = dinv_ref[...] * tot + b_ref[...]


def _sc_mesh():
    return plsc.VectorSubcoreMesh(core_axis_name="c", subcore_axis_name="s")


def _deg_call(fei, clus):
    f = pl.kernel(
        _deg_body,
        out_type=(
            jax.ShapeDtypeStruct((_NW, _N), jnp.float32),
            jax.ShapeDtypeStruct((_NW, _RSZ), jnp.int32),
            jax.ShapeDtypeStruct((_NW, _RSZ), jnp.int32),
            jax.ShapeDtypeStruct((_NW, _L), jnp.int32),
        ),
        mesh=_sc_mesh(),
        scratch_types=[
            pltpu.VMEM((_EPT,), jnp.int32),
            pltpu.VMEM((_EPT,), jnp.int32),
            pltpu.VMEM((_N,), jnp.int32),
            pltpu.VMEM((_N,), jnp.float32),
            pltpu.VMEM((_RSZ + _L,), jnp.int32),
            pltpu.VMEM((_RSZ + _L,), jnp.int32),
            pltpu.VMEM((_L,), jnp.int32),
        ],
        compiler_params=pltpu.CompilerParams(
            needs_layout_passes=False, use_tc_tiling_on_sc=False),
    )
    return f(fei, clus)


def _agg_call(y2, csrc_a, cdst_a, cnt):
    f = pl.kernel(
        _agg_body,
        out_type=jax.ShapeDtypeStruct((_NC, _N_PAD, _DH), jnp.float32),
        mesh=_sc_mesh(),
        scratch_types=[
            pltpu.VMEM((2, _CREG, _CWA), jnp.int32),
            pltpu.VMEM((2, _CREG, _CWA), jnp.int32),
            pltpu.VMEM((2, _L), jnp.int32),
            pltpu.VMEM((_CWA, _DH), jnp.float32),
            pltpu.VMEM((_CWA, _DH), jnp.float32),
            pltpu.VMEM((16, _DH), jnp.float32),
            pltpu.VMEM_SHARED((_N_PAD, _DH), jnp.float32),
            pltpu.SemaphoreType.DMA,
            pltpu.SemaphoreType.DMA,
        ],
        compiler_params=pltpu.CompilerParams(
            needs_layout_passes=False, use_tc_tiling_on_sc=False),
    )
    return f(y2, csrc_a, cdst_a, cnt)


def _dinv_call(hist):
    return pl.pallas_call(
        _dinv_body,
        out_shape=jax.ShapeDtypeStruct((_N, 1), jnp.float32),
    )(hist)


def _y_call(x, w, dinv):
    return pl.pallas_call(
        _y_body,
        grid=(_N // _BR,),
        in_specs=[
            pl.BlockSpec((_BR, _D), lambda i: (i, 0)),
            pl.BlockSpec((_D, _D), lambda i: (0, 0)),
            pl.BlockSpec((_BR, 1), lambda i: (i, 0)),
        ],
        out_specs=pl.BlockSpec((_NC, _BR, _DH), lambda i: (0, i, 0)),
        out_shape=jax.ShapeDtypeStruct((_NC, _N, _DH), jnp.float32),
    )(x, w, dinv)


def _combine_call(agg, y2, dinv, b2):
    return pl.pallas_call(
        _combine_body,
        grid=(_N // _BR,),
        in_specs=[
            pl.BlockSpec((_NC, _BR, _DH), lambda i: (0, i, 0)),
            pl.BlockSpec((_NC, _BR, _DH), lambda i: (0, i, 0)),
            pl.BlockSpec((_BR, 1), lambda i: (i, 0)),
            pl.BlockSpec((1, _D), lambda i: (0, 0)),
        ],
        out_specs=pl.BlockSpec((_BR, _D), lambda i: (i, 0)),
        out_shape=jax.ShapeDtypeStruct((_N, _D), jnp.float32),
    )(agg, y2, dinv, b2)


def kernel(X, W, b, cluster_assignment, full_edge_index):
    n, d = X.shape
    hist, csrc, cdst, cnt = _deg_call(full_edge_index, cluster_assignment)
    dinv = _dinv_call(hist)
    y2 = _y_call(X, W, dinv)
    agg = _agg_call(y2,
                    csrc.reshape(_NW, _CREG, _CWA),
                    cdst.reshape(_NW, _CREG, _CWA),
                    cnt.reshape(_NS, 2, _L))
    return _combine_call(agg, y2, dinv, b.reshape(1, d))
